# baseline (speedup 1.0000x reference)
import functools

import jax
import jax.numpy as jnp
from jax.experimental import pallas as pl
from jax.experimental.pallas import tpu as pltpu

_INTERPRET = False


def _pick_bm(n):
    for bm in (200, 400, 256, 128, 1000, 64, 32, 16, 8):
        if n % bm == 0:
            return bm
    return n


def _softmax_attn(h, wf1t, wf2t):
    a = jnp.dot(h, wf1t, preferred_element_type=jnp.float32)
    a = jnp.dot(a, wf2t, preferred_element_type=jnp.float32)
    a = a - jnp.max(a, axis=1, keepdims=True)
    e = jnp.exp(a)
    return h * (e / jnp.sum(e, axis=1, keepdims=True))


def _big_dot(adj_tile, s):
    return jnp.dot(adj_tile, s, preferred_element_type=jnp.float32)


def _mid_body(adj_ref, x_ref, w1t_ref, b1_ref, wf1t_ref, wf2t_ref, wnt_ref,
              bn_ref, out_ref, s_scr):
    @pl.when(pl.program_id(0) == 0)
    def _():
        s_scr[...] = (
            jnp.dot(x_ref[...], w1t_ref[...], preferred_element_type=jnp.float32)
            + b1_ref[...]
        )

    h = jnp.maximum(_big_dot(adj_ref[...], s_scr[...]), 0.0)
    h = _softmax_attn(h, wf1t_ref[...], wf2t_ref[...])
    out_ref[...] = (
        jnp.dot(h, wnt_ref[...], preferred_element_type=jnp.float32) + bn_ref[...]
    )


def _last_body(adj_ref, s_ref, wf1t_ref, wf2t_ref, out_ref):
    h = jnp.maximum(_big_dot(adj_ref[...], s_ref[...]), 0.0)
    out_ref[...] = _softmax_attn(h, wf1t_ref[...], wf2t_ref[...])


def _final_body(adj_ref, s_ref, wf1t_ref, wf2t_ref, h1_ref, wfa_ref, wfb_ref,
                bfc_ref, out_ref):
    h2 = jnp.maximum(_big_dot(adj_ref[...], s_ref[...]), 0.0)
    h2 = _softmax_attn(h2, wf1t_ref[...], wf2t_ref[...])
    out_ref[...] = (
        jnp.dot(h1_ref[...], wfa_ref[...], preferred_element_type=jnp.float32)
        + jnp.dot(h2, wfb_ref[...], preferred_element_type=jnp.float32)
        + bfc_ref[...]
    )


def _full_spec(arr):
    return pl.BlockSpec(arr.shape, lambda i: (0,) * arr.ndim)


def _row_spec(bm, ncol):
    return pl.BlockSpec((bm, ncol), lambda i: (i, 0))


def _mid_layer(adj, x, w1t, b1, wf1t, wf2t, wnt, bn):
    n = adj.shape[0]
    bm = _pick_bm(n)
    return pl.pallas_call(
        _mid_body,
        grid=(n // bm,),
        in_specs=[
            _row_spec(bm, adj.shape[1]),
            _full_spec(x),
            _full_spec(w1t),
            _full_spec(b1),
            _full_spec(wf1t),
            _full_spec(wf2t),
            _full_spec(wnt),
            _full_spec(bn),
        ],
        out_specs=_row_spec(bm, wnt.shape[1]),
        out_shape=jax.ShapeDtypeStruct((n, wnt.shape[1]), jnp.float32),
        scratch_shapes=[pltpu.VMEM((adj.shape[1], w1t.shape[1]), jnp.float32)],
        compiler_params=pltpu.CompilerParams(
            dimension_semantics=("arbitrary",)),
        interpret=_INTERPRET,
    )(adj, x, w1t, b1, wf1t, wf2t, wnt, bn)


def _last_layer(adj, s, wf1t, wf2t):
    n = adj.shape[0]
    c = s.shape[1]
    bm = _pick_bm(n)
    return pl.pallas_call(
        _last_body,
        grid=(n // bm,),
        in_specs=[
            _row_spec(bm, adj.shape[1]),
            _full_spec(s),
            _full_spec(wf1t),
            _full_spec(wf2t),
        ],
        out_specs=_row_spec(bm, c),
        out_shape=jax.ShapeDtypeStruct((n, c), jnp.float32),
        compiler_params=pltpu.CompilerParams(
            dimension_semantics=("arbitrary",)),
        interpret=_INTERPRET,
    )(adj, s, wf1t, wf2t)


def _final_layer(adj, s, wf1t, wf2t, h1, wfa, wfb, bfc):
    n = adj.shape[0]
    nclass = wfa.shape[1]
    bm = _pick_bm(n)
    return pl.pallas_call(
        _final_body,
        grid=(n // bm,),
        in_specs=[
            _row_spec(bm, adj.shape[1]),
            _full_spec(s),
            _full_spec(wf1t),
            _full_spec(wf2t),
            _row_spec(bm, h1.shape[1]),
            _full_spec(wfa),
            _full_spec(wfb),
            _full_spec(bfc),
        ],
        out_specs=_row_spec(bm, nclass),
        out_shape=jax.ShapeDtypeStruct((n, nclass), jnp.float32),
        compiler_params=pltpu.CompilerParams(
            dimension_semantics=("arbitrary",)),
        interpret=_INTERPRET,
    )(adj, s, wf1t, wf2t, h1, wfa, wfb, bfc)


def kernel(x1, adj1, x2, adj2, W1, b1, W2, b2, Wa1_1, Wa1_2, Wa2_1, Wa2_2,
           Wfc, bfc):
    w1t = W1.T
    w2t = W2.T
    wa11t = Wa1_1.T
    wa12t = Wa1_2.T
    wa21t = Wa2_1.T
    wa22t = Wa2_2.T
    b1r = b1.reshape(1, -1)
    b2r = b2.reshape(1, -1)
    bfcr = bfc.reshape(1, -1)
    nout = W2.shape[0]
    wfct = Wfc.T
    wfa = wfct[:nout]
    wfb = wfct[nout:]

    t1 = _mid_layer(adj1, x1, w1t, b1r, wa11t, wa12t, w2t, b2r)
    h1 = _last_layer(adj1, t1, wa21t, wa22t)

    t2 = _mid_layer(adj2, x2, w1t, b1r, wa11t, wa12t, w2t, b2r)
    return _final_layer(adj2, t2, wa21t, wa22t, h1, wfa, wfb, bfcr)


# speedup vs baseline: 1.1106x; 1.1106x over previous
import jax
import jax.numpy as jnp
from jax.experimental import pallas as pl
from jax.experimental.pallas import tpu as pltpu

_INTERPRET = False


def _pick_bm(n):
    for bm in (400, 256, 200, 128, 64, 32, 16, 8):
        if n % bm == 0:
            return bm
    return n


def _softmax_attn(h, wf1t, wf2t):
    a = jnp.dot(h, wf1t, preferred_element_type=jnp.float32)
    a = jnp.dot(a, wf2t, preferred_element_type=jnp.float32)
    a = a - jnp.max(a, axis=1, keepdims=True)
    e = jnp.exp(a)
    return h * (e / jnp.sum(e, axis=1, keepdims=True))


def _big_dot(adj_tile, s):
    return jnp.dot(adj_tile, s, preferred_element_type=jnp.float32)


def _branch1_body(adj_ref, x_ref, w1t_ref, b1_ref, wa11t_ref, wa12t_ref,
                  w2t_ref, b2_ref, wa21t_ref, wa22t_ref, out_ref, s_scr,
                  t_scr):
    stage = pl.program_id(0)
    i = pl.program_id(1)
    bm = adj_ref.shape[0]

    @pl.when((stage == 0) & (i == 0))
    def _():
        s_scr[...] = (
            jnp.dot(x_ref[...], w1t_ref[...], preferred_element_type=jnp.float32)
            + b1_ref[...]
        )

    @pl.when(stage == 0)
    def _():
        h = jnp.maximum(_big_dot(adj_ref[...], s_scr[...]), 0.0)
        h = _softmax_attn(h, wa11t_ref[...], wa12t_ref[...])
        t_scr[pl.ds(i * bm, bm), :] = (
            jnp.dot(h, w2t_ref[...], preferred_element_type=jnp.float32)
            + b2_ref[...]
        )

    @pl.when(stage == 1)
    def _():
        h = jnp.maximum(_big_dot(adj_ref[...], t_scr[...]), 0.0)
        out_ref[...] = _softmax_attn(h, wa21t_ref[...], wa22t_ref[...])


def _branch2_body(adj_ref, x_ref, h1_ref, w1t_ref, b1_ref, wa11t_ref,
                  wa12t_ref, w2t_ref, b2_ref, wa21t_ref, wa22t_ref, wfa_ref,
                  wfb_ref, bfc_ref, out_ref, s_scr, t_scr):
    stage = pl.program_id(0)
    i = pl.program_id(1)
    bm = adj_ref.shape[0]

    @pl.when((stage == 0) & (i == 0))
    def _():
        s_scr[...] = (
            jnp.dot(x_ref[...], w1t_ref[...], preferred_element_type=jnp.float32)
            + b1_ref[...]
        )

    @pl.when(stage == 0)
    def _():
        h = jnp.maximum(_big_dot(adj_ref[...], s_scr[...]), 0.0)
        h = _softmax_attn(h, wa11t_ref[...], wa12t_ref[...])
        t_scr[pl.ds(i * bm, bm), :] = (
            jnp.dot(h, w2t_ref[...], preferred_element_type=jnp.float32)
            + b2_ref[...]
        )

    @pl.when(stage == 1)
    def _():
        h2 = jnp.maximum(_big_dot(adj_ref[...], t_scr[...]), 0.0)
        h2 = _softmax_attn(h2, wa21t_ref[...], wa22t_ref[...])
        out_ref[...] = (
            jnp.dot(h1_ref[...], wfa_ref[...], preferred_element_type=jnp.float32)
            + jnp.dot(h2, wfb_ref[...], preferred_element_type=jnp.float32)
            + bfc_ref[...]
        )


def _full_spec(arr):
    return pl.BlockSpec(arr.shape, lambda s, i: (0,) * arr.ndim)


def _row_spec(bm, ncol):
    return pl.BlockSpec((bm, ncol), lambda s, i: (i, 0))


def _branch1(adj, x, w1t, b1, wa11t, wa12t, w2t, b2, wa21t, wa22t):
    n = adj.shape[0]
    c = w2t.shape[1]
    bm = _pick_bm(n)
    return pl.pallas_call(
        _branch1_body,
        grid=(2, n // bm),
        in_specs=[
            _row_spec(bm, adj.shape[1]),
            _full_spec(x),
            _full_spec(w1t),
            _full_spec(b1),
            _full_spec(wa11t),
            _full_spec(wa12t),
            _full_spec(w2t),
            _full_spec(b2),
            _full_spec(wa21t),
            _full_spec(wa22t),
        ],
        out_specs=_row_spec(bm, c),
        out_shape=jax.ShapeDtypeStruct((n, c), jnp.float32),
        scratch_shapes=[
            pltpu.VMEM((adj.shape[1], w1t.shape[1]), jnp.float32),
            pltpu.VMEM((n, c), jnp.float32),
        ],
        compiler_params=pltpu.CompilerParams(
            dimension_semantics=("arbitrary", "arbitrary")),
        interpret=_INTERPRET,
    )(adj, x, w1t, b1, wa11t, wa12t, w2t, b2, wa21t, wa22t)


def _branch2(adj, x, h1, w1t, b1, wa11t, wa12t, w2t, b2, wa21t, wa22t, wfa,
             wfb, bfc):
    n = adj.shape[0]
    c = w2t.shape[1]
    nclass = wfa.shape[1]
    bm = _pick_bm(n)
    return pl.pallas_call(
        _branch2_body,
        grid=(2, n // bm),
        in_specs=[
            _row_spec(bm, adj.shape[1]),
            _full_spec(x),
            _row_spec(bm, h1.shape[1]),
            _full_spec(w1t),
            _full_spec(b1),
            _full_spec(wa11t),
            _full_spec(wa12t),
            _full_spec(w2t),
            _full_spec(b2),
            _full_spec(wa21t),
            _full_spec(wa22t),
            _full_spec(wfa),
            _full_spec(wfb),
            _full_spec(bfc),
        ],
        out_specs=_row_spec(bm, nclass),
        out_shape=jax.ShapeDtypeStruct((n, nclass), jnp.float32),
        scratch_shapes=[
            pltpu.VMEM((adj.shape[1], w1t.shape[1]), jnp.float32),
            pltpu.VMEM((n, c), jnp.float32),
        ],
        compiler_params=pltpu.CompilerParams(
            dimension_semantics=("arbitrary", "arbitrary")),
        interpret=_INTERPRET,
    )(adj, x, h1, w1t, b1, wa11t, wa12t, w2t, b2, wa21t, wa22t, wfa, wfb, bfc)


def kernel(x1, adj1, x2, adj2, W1, b1, W2, b2, Wa1_1, Wa1_2, Wa2_1, Wa2_2,
           Wfc, bfc):
    w1t = W1.T
    w2t = W2.T
    wa11t = Wa1_1.T
    wa12t = Wa1_2.T
    wa21t = Wa2_1.T
    wa22t = Wa2_2.T
    b1r = b1.reshape(1, -1)
    b2r = b2.reshape(1, -1)
    bfcr = bfc.reshape(1, -1)
    nout = W2.shape[0]
    wfct = Wfc.T
    wfa = wfct[:nout]
    wfb = wfct[nout:]

    h1 = _branch1(adj1, x1, w1t, b1r, wa11t, wa12t, w2t, b2r, wa21t, wa22t)
    return _branch2(adj2, x2, h1, w1t, b1r, wa11t, wa12t, w2t, b2r, wa21t,
                    wa22t, wfa, wfb, bfcr)
